# Initial kernel scaffold; baseline (speedup 1.0000x reference)
import jax
import jax.numpy as jnp
from jax import lax
from jax.experimental import pallas as pl
from jax.experimental.pallas import tpu as pltpu
from jax.experimental.pallas import tpu_sc as plsc

F = 128
NC = 2
NS = 16


def _make_sc_agg(n, e, n_chunks, with_counts):
    rpt = n // NS
    ept = e // NS
    k = 80
    nb = ept // k
    ept2 = e // (NS * NC)
    k2 = 40
    nb2 = ept2 // k2
    assert ept % k == 0 and ept2 % k2 == 0 and n % NS == 0

    mesh = plsc.VectorSubcoreMesh(core_axis_name="c", subcore_axis_name="s")

    out_type = [jax.ShapeDtypeStruct((n_chunks, n, F), jnp.float32)]
    if with_counts:
        out_type.append(jax.ShapeDtypeStruct((NC, n, 16), jnp.float32))

    scratch = [
        pltpu.VMEM_SHARED((n, F), jnp.float32),
        pltpu.VMEM((k,), jnp.int32),
        pltpu.VMEM((k,), jnp.int32),
        pltpu.VMEM((k, F), jnp.float32),
        pltpu.SemaphoreType.DMA,
    ]
    if with_counts:
        scratch += [
            pltpu.VMEM_SHARED((n, 16), jnp.float32),
            pltpu.VMEM((k2,), jnp.int32),
            pltpu.VMEM((k2, 16), jnp.float32),
        ]

    def body(*refs):
        i = 0
        tables = refs[:n_chunks]; i += n_chunks
        src = refs[i]; dst = refs[i + 1]; zeros = refs[i + 2]; i += 3
        if with_counts:
            zeros16 = refs[i]; ones_h = refs[i + 1]; i += 2
        out = refs[i]; i += 1
        if with_counts:
            out_cnt = refs[i]; i += 1
        acc = refs[i]; sidx = refs[i + 1]; didx = refs[i + 2]
        rows = refs[i + 3]; sem = refs[i + 4]; i += 5
        if with_counts:
            cacc = refs[i]; didx2 = refs[i + 1]; onesv = refs[i + 2]

        cid = lax.axis_index("c")
        sid = lax.axis_index("s")

        if with_counts:
            pltpu.sync_copy(zeros16, cacc.at[pl.ds(sid * rpt, rpt)])
            pltpu.sync_copy(ones_h, onesv)
            plsc.subcore_barrier()
            cbase = (cid * NS + sid) * ept2

            def cstep(b, carry):
                off = cbase + b * k2
                pltpu.sync_copy(dst.at[pl.ds(off, k2)], didx2)
                pltpu.sync_copy(onesv, cacc.at[didx2], add=True)
                return carry

            lax.fori_loop(0, nb2, cstep, 0)
            plsc.subcore_barrier()
            pltpu.sync_copy(cacc.at[pl.ds(sid * rpt, rpt)],
                            out_cnt.at[cid, pl.ds(sid * rpt, rpt)])

        for c in range(n_chunks):
            @pl.when(cid == (c % NC))
            def _(c=c):
                pltpu.sync_copy(zeros, acc.at[pl.ds(sid * rpt, rpt)])
                plsc.subcore_barrier()
                ebase = sid * ept
                tab = tables[c]

                def estep(b, carry):
                    off = ebase + b * k
                    pltpu.sync_copy(src.at[pl.ds(off, k)], sidx)
                    pltpu.sync_copy(dst.at[pl.ds(off, k)], didx)
                    pltpu.async_copy(tab.at[sidx], rows, sem).wait()
                    pltpu.sync_copy(rows, acc.at[didx], add=True)
                    return carry

                lax.fori_loop(0, nb, estep, 0)
                plsc.subcore_barrier()
                pltpu.sync_copy(acc.at[pl.ds(sid * rpt, rpt)],
                                out.at[c, pl.ds(sid * rpt, rpt)])

    return pl.kernel(body, out_type=out_type, mesh=mesh, scratch_types=scratch)


def _tc_layer(aggr, cnt, h_chunks, Wl, Wr, bl, relu, blk):
    c_in = aggr.shape[0]
    n = aggr.shape[1]
    h_dim = Wl.shape[1]
    c_out = h_dim // F
    nblk = n // blk

    def body(*refs):
        agg_ref = refs[0]
        cnt_ref = refs[1]
        h_refs = refs[2:2 + c_in]
        wl_ref = refs[2 + c_in]
        wr_ref = refs[3 + c_in]
        bl_ref = refs[4 + c_in]
        out_refs = refs[5 + c_in:]

        cnt2 = cnt_ref[0, :, :1] + cnt_ref[1, :, :1]
        inv = 1.0 / jnp.maximum(cnt2, 1.0)
        acc = jnp.broadcast_to(bl_ref[...], (blk, h_dim))
        for c in range(c_in):
            a = agg_ref[c] * inv
            acc = acc + jnp.dot(a, wl_ref[c * F:(c + 1) * F, :],
                                preferred_element_type=jnp.float32)
            acc = acc + jnp.dot(h_refs[c][...], wr_ref[c * F:(c + 1) * F, :],
                                preferred_element_type=jnp.float32)
        if relu:
            acc = jnp.maximum(acc, 0.0)
        for j in range(c_out):
            out_refs[j][...] = acc[:, j * F:(j + 1) * F]

    in_specs = [
        pl.BlockSpec((c_in, blk, F), lambda i: (0, i, 0)),
        pl.BlockSpec((NC, blk, 16), lambda i: (0, i, 0)),
    ]
    in_specs += [pl.BlockSpec((blk, F), lambda i: (i, 0)) for _ in range(c_in)]
    in_specs += [
        pl.BlockSpec((c_in * F, h_dim), lambda i: (0, 0)),
        pl.BlockSpec((c_in * F, h_dim), lambda i: (0, 0)),
        pl.BlockSpec((1, h_dim), lambda i: (0, 0)),
    ]
    out_specs = [pl.BlockSpec((blk, F), lambda i: (i, 0)) for _ in range(c_out)]
    out_shape = [jax.ShapeDtypeStruct((n, F), jnp.float32) for _ in range(c_out)]

    return pl.pallas_call(
        body, grid=(nblk,), in_specs=in_specs, out_specs=out_specs,
        out_shape=out_shape,
    )(aggr, cnt, *h_chunks, Wl, Wr, bl)


def _tc_pool_classifier(h_chunks, batch_r, Wc1, bc1, Wc2, bc2, g, blk):
    n = h_chunks[0].shape[0]
    nblk = n // blk
    h_dim = F * len(h_chunks)
    hid = Wc1.shape[1]
    n_cls = Wc2.shape[1]

    def body(*refs):
        h_refs = refs[:4]
        b_ref = refs[4]
        wc1_ref, bc1_ref, wc2_ref, bc2_ref = refs[5:9]
        logits_ref, emb_ref = refs[9], refs[10]
        gsum, gcnt = refs[11], refs[12]

        i = pl.program_id(0)

        @pl.when(i == 0)
        def _():
            gsum[...] = jnp.zeros_like(gsum)
            gcnt[...] = jnp.zeros_like(gcnt)

        bids = b_ref[0]
        iot = lax.broadcasted_iota(jnp.int32, (g, blk), 0)
        oh = (iot == bids).astype(jnp.float32)
        for c in range(4):
            gsum[:, c * F:(c + 1) * F] += jnp.dot(
                oh, h_refs[c][...], preferred_element_type=jnp.float32)
        gcnt[...] += jnp.broadcast_to(
            jnp.sum(oh, axis=1, keepdims=True), (g, h_dim))

        @pl.when(i == nblk - 1)
        def _():
            emb = gsum[...] * (1.0 / jnp.maximum(gcnt[...], 1.0))
            hc = jnp.dot(emb, wc1_ref[...], preferred_element_type=jnp.float32)
            hc = jnp.maximum(hc + bc1_ref[...], 0.0)
            logits_ref[...] = jnp.dot(
                hc, wc2_ref[...], preferred_element_type=jnp.float32) + bc2_ref[...]
            emb_ref[...] = emb

    in_specs = [pl.BlockSpec((blk, F), lambda i: (i, 0)) for _ in range(4)]
    in_specs += [
        pl.BlockSpec((1, 1, blk), lambda i: (i, 0, 0)),
        pl.BlockSpec((h_dim, hid), lambda i: (0, 0)),
        pl.BlockSpec((1, hid), lambda i: (0, 0)),
        pl.BlockSpec((hid, n_cls), lambda i: (0, 0)),
        pl.BlockSpec((1, n_cls), lambda i: (0, 0)),
    ]
    out_specs = [
        pl.BlockSpec((g, n_cls), lambda i: (0, 0)),
        pl.BlockSpec((g, h_dim), lambda i: (0, 0)),
    ]
    out_shape = [
        jax.ShapeDtypeStruct((g, n_cls), jnp.float32),
        jax.ShapeDtypeStruct((g, h_dim), jnp.float32),
    ]
    return pl.pallas_call(
        body, grid=(nblk,), in_specs=in_specs, out_specs=out_specs,
        out_shape=out_shape,
        scratch_shapes=[
            pltpu.VMEM((g, h_dim), jnp.float32),
            pltpu.VMEM((g, h_dim), jnp.float32),
        ],
    )(*h_chunks, batch_r, Wc1, bc1, Wc2, bc2)


def kernel(x, edge_index, batch, Wl0, bl0, Wr0, Wl1, bl1, Wr1,
           Wl2, bl2, Wr2, Wc1, bc1, Wc2, bc2):
    n, d_in = x.shape
    e = edge_index.shape[1]
    h_dim = Wl0.shape[1]
    g = Wc1.shape[0] and 64
    blk = 1000

    src = edge_index[0]
    dst = edge_index[1]

    rpt = n // NS
    zeros = jnp.zeros((rpt, F), jnp.float32)
    zeros16 = jnp.zeros((rpt, 16), jnp.float32)
    ones = jnp.ones((40, 16), jnp.float32)

    x_chunks = [x[:, c * F:(c + 1) * F] for c in range(d_in // F)]
    batch_r = batch.reshape(n // 2000, 1, 2000)

    sc_l0 = _make_sc_agg(n, e, d_in // F, True)
    sc_l12 = _make_sc_agg(n, e, h_dim // F, False)

    agg0, cnt = sc_l0(*x_chunks, src, dst, zeros, zeros16, ones)
    h1 = _tc_layer(agg0, cnt, x_chunks, Wl0, Wr0,
                   bl0.reshape(1, -1), True, blk)
    agg1 = sc_l12(*h1, src, dst, zeros)
    h2 = _tc_layer(agg1, cnt, h1, Wl1, Wr1, bl1.reshape(1, -1), True, blk)
    agg2 = sc_l12(*h2, src, dst, zeros)
    h3 = _tc_layer(agg2, cnt, h2, Wl2, Wr2, bl2.reshape(1, -1), False, blk)

    logits, emb = _tc_pool_classifier(
        h3, batch_r, Wc1, bc1.reshape(1, -1), Wc2, bc2.reshape(1, -1),
        g, 2000)
    return (logits, emb)


# speedup vs baseline: 3.0653x; 3.0653x over previous
import jax
import jax.numpy as jnp
from jax import lax
from jax.experimental import pallas as pl
from jax.experimental.pallas import tpu as pltpu
from jax.experimental.pallas import tpu_sc as plsc

F = 128
NC = 2
NS = 16


def _make_sc_agg(n, npad, e, n_chunks, with_counts):
    rpt = npad // NS
    ept = e // NS
    k = 80
    nb = ept // k
    ept2 = e // (NS * NC)
    k2 = 40
    nb2 = ept2 // k2
    assert ept % k == 0 and ept2 % k2 == 0 and npad % (8 * NS) == 0
    assert n_chunks % NC == 0

    mesh = plsc.VectorSubcoreMesh(core_axis_name="c", subcore_axis_name="s")

    out_type = [jax.ShapeDtypeStruct((n_chunks * npad, F), jnp.float32)]
    if with_counts:
        out_type.append(jax.ShapeDtypeStruct((NC * npad, F), jnp.float32))

    scratch = [
        pltpu.VMEM_SHARED((npad, F), jnp.float32),
        pltpu.VMEM((k,), jnp.int32),
        pltpu.VMEM((k,), jnp.int32),
        pltpu.VMEM((k, F), jnp.float32),
        pltpu.SemaphoreType.DMA,
    ]
    if with_counts:
        scratch += [pltpu.VMEM((k2,), jnp.int32)]

    def body(*refs):
        i = 0
        tab = refs[0]
        src_all = refs[1]
        dst = refs[2]
        zeros = refs[3]; i = 4
        if with_counts:
            ones_h = refs[i]; i += 1
        out = refs[i]; i += 1
        if with_counts:
            out_cnt = refs[i]; i += 1
        acc = refs[i]; sidx = refs[i + 1]; didx = refs[i + 2]
        rows = refs[i + 3]; sem = refs[i + 4]; i += 5
        if with_counts:
            didx2 = refs[i]

        cid = lax.axis_index("c")
        sid = lax.axis_index("s")

        for j in range(n_chunks // NC):
            chunk = j * NC + cid
            pltpu.sync_copy(zeros, acc.at[pl.ds(sid * rpt, rpt)])
            plsc.subcore_barrier()
            ebase = chunk * e + sid * ept

            def estep(b, carry):
                off = ebase + b * k
                pltpu.sync_copy(src_all.at[pl.ds(off, k)], sidx)
                pltpu.sync_copy(dst.at[pl.ds(sid * ept + b * k, k)], didx)
                pltpu.async_copy(tab.at[sidx], rows, sem).wait()
                pltpu.sync_copy(rows, acc.at[didx], add=True)
                return carry

            lax.fori_loop(0, nb, estep, 0)
            plsc.subcore_barrier()
            pltpu.sync_copy(acc.at[pl.ds(sid * rpt, rpt)],
                            out.at[pl.ds(chunk * npad + sid * rpt, rpt)])
            plsc.subcore_barrier()

        if with_counts:
            pltpu.sync_copy(zeros, acc.at[pl.ds(sid * rpt, rpt)])
            pltpu.sync_copy(ones_h, rows.at[pl.ds(0, k2)])
            plsc.subcore_barrier()
            cbase = (cid * NS + sid) * ept2

            def cstep(b, carry):
                off = cbase + b * k2
                pltpu.sync_copy(dst.at[pl.ds(off, k2)], didx2)
                pltpu.sync_copy(rows.at[pl.ds(0, k2)], acc.at[didx2], add=True)
                return carry

            lax.fori_loop(0, nb2, cstep, 0)
            plsc.subcore_barrier()
            pltpu.sync_copy(acc.at[pl.ds(sid * rpt, rpt)],
                            out_cnt.at[pl.ds(cid * npad + sid * rpt, rpt)])

    return pl.kernel(body, out_type=out_type, mesh=mesh, scratch_types=scratch)


def _tc_layer(aggr, cnt, h_r, Wl, Wr, bl, relu, blk):
    c_in, _, _ = aggr.shape
    n = h_r.shape[1]
    h_dim = Wl.shape[1]
    c_out = h_dim // F
    nblk = n // blk

    def body(agg_ref, cnt_ref, h_ref, wl_ref, wr_ref, bl_ref, out_ref):
        cnt2 = cnt_ref[0, :, :1] + cnt_ref[1, :, :1]
        inv = 1.0 / jnp.maximum(cnt2, 1.0)
        acc = jnp.broadcast_to(bl_ref[...], (blk, h_dim))
        for c in range(c_in):
            a = agg_ref[c] * inv
            acc = acc + jnp.dot(a, wl_ref[c * F:(c + 1) * F, :],
                                preferred_element_type=jnp.float32)
            acc = acc + jnp.dot(h_ref[c], wr_ref[c * F:(c + 1) * F, :],
                                preferred_element_type=jnp.float32)
        if relu:
            acc = jnp.maximum(acc, 0.0)
        for j in range(c_out):
            out_ref[j] = acc[:, j * F:(j + 1) * F]

    in_specs = [
        pl.BlockSpec((c_in, blk, F), lambda i: (0, i, 0)),
        pl.BlockSpec((NC, blk, F), lambda i: (0, i, 0)),
        pl.BlockSpec((c_in, blk, F), lambda i: (0, i, 0)),
        pl.BlockSpec((c_in * F, h_dim), lambda i: (0, 0)),
        pl.BlockSpec((c_in * F, h_dim), lambda i: (0, 0)),
        pl.BlockSpec((1, h_dim), lambda i: (0, 0)),
    ]
    out_specs = pl.BlockSpec((c_out, blk, F), lambda i: (0, i, 0))
    out_shape = jax.ShapeDtypeStruct((c_out, n, F), jnp.float32)

    return pl.pallas_call(
        body, grid=(nblk,), in_specs=in_specs, out_specs=out_specs,
        out_shape=out_shape,
    )(aggr, cnt, h_r, Wl, Wr, bl)


def _tc_pool_classifier(h_r, batch_r, Wc1, bc1, Wc2, bc2, g, blk):
    n = h_r.shape[1]
    nblk = n // blk
    h_dim = F * h_r.shape[0]
    hid = Wc1.shape[1]
    n_cls = Wc2.shape[1]

    def body(h_ref, b_ref, wc1_ref, bc1_ref, wc2_ref, bc2_ref,
             logits_ref, emb_ref, gsum, gcnt):
        i = pl.program_id(0)

        @pl.when(i == 0)
        def _():
            gsum[...] = jnp.zeros_like(gsum)
            gcnt[...] = jnp.zeros_like(gcnt)

        bids = b_ref[0]
        iot = lax.broadcasted_iota(jnp.int32, (g, blk), 0)
        oh = (iot == bids).astype(jnp.float32)
        for c in range(4):
            gsum[:, c * F:(c + 1) * F] += jnp.dot(
                oh, h_ref[c], preferred_element_type=jnp.float32)
        gcnt[...] += jnp.broadcast_to(
            jnp.sum(oh, axis=1, keepdims=True), (g, h_dim))

        @pl.when(i == nblk - 1)
        def _():
            emb = gsum[...] * (1.0 / jnp.maximum(gcnt[...], 1.0))
            hc = jnp.dot(emb, wc1_ref[...], preferred_element_type=jnp.float32)
            hc = jnp.maximum(hc + bc1_ref[...], 0.0)
            logits_ref[...] = jnp.dot(
                hc, wc2_ref[...], preferred_element_type=jnp.float32) + bc2_ref[...]
            emb_ref[...] = emb

    in_specs = [
        pl.BlockSpec((4, blk, F), lambda i: (0, i, 0)),
        pl.BlockSpec((1, 1, blk), lambda i: (i, 0, 0)),
        pl.BlockSpec((h_dim, hid), lambda i: (0, 0)),
        pl.BlockSpec((1, hid), lambda i: (0, 0)),
        pl.BlockSpec((hid, n_cls), lambda i: (0, 0)),
        pl.BlockSpec((1, n_cls), lambda i: (0, 0)),
    ]
    out_specs = [
        pl.BlockSpec((g, n_cls), lambda i: (0, 0)),
        pl.BlockSpec((g, h_dim), lambda i: (0, 0)),
    ]
    out_shape = [
        jax.ShapeDtypeStruct((g, n_cls), jnp.float32),
        jax.ShapeDtypeStruct((g, h_dim), jnp.float32),
    ]
    return pl.pallas_call(
        body, grid=(nblk,), in_specs=in_specs, out_specs=out_specs,
        out_shape=out_shape,
        scratch_shapes=[
            pltpu.VMEM((g, h_dim), jnp.float32),
            pltpu.VMEM((g, h_dim), jnp.float32),
        ],
    )(h_r, batch_r, Wc1, bc1, Wc2, bc2)


def kernel(x, edge_index, batch, Wl0, bl0, Wr0, Wl1, bl1, Wr1,
           Wl2, bl2, Wr2, Wc1, bc1, Wc2, bc2):
    n, d_in = x.shape
    e = edge_index.shape[1]
    h_dim = Wl0.shape[1]
    g = 64
    blk = 1000
    c0 = d_in // F
    c1 = h_dim // F

    src = edge_index[0]
    dst = edge_index[1]
    src2 = (jnp.arange(c0, dtype=jnp.int32)[:, None] * n + src[None, :]).reshape(-1)
    src4 = (jnp.arange(c1, dtype=jnp.int32)[:, None] * n + src[None, :]).reshape(-1)

    npad = 10240
    rpt = npad // NS
    zeros = jnp.zeros((rpt, F), jnp.float32)
    ones = jnp.ones((40, F), jnp.float32)

    x_r = x.reshape(n, c0, F).transpose(1, 0, 2)
    batch_r = batch.reshape(n // 2000, 1, 2000)

    sc_l0 = _make_sc_agg(n, npad, e, c0, True)
    sc_l12 = _make_sc_agg(n, npad, e, c1, False)

    agg0, cnt = sc_l0(x_r.reshape(c0 * n, F), src2, dst, zeros, ones)
    agg0 = agg0.reshape(c0, npad, F)
    cnt = cnt.reshape(NC, npad, F)
    h1 = _tc_layer(agg0, cnt, x_r, Wl0, Wr0, bl0.reshape(1, -1), True, blk)
    [agg1] = sc_l12(h1.reshape(c1 * n, F), src4, dst, zeros)
    h2 = _tc_layer(agg1.reshape(c1, npad, F), cnt, h1, Wl1, Wr1,
                   bl1.reshape(1, -1), True, blk)
    [agg2] = sc_l12(h2.reshape(c1 * n, F), src4, dst, zeros)
    h3 = _tc_layer(agg2.reshape(c1, npad, F), cnt, h2, Wl2, Wr2,
                   bl2.reshape(1, -1), False, blk)

    logits, emb = _tc_pool_classifier(
        h3, batch_r, Wc1, bc1.reshape(1, -1), Wc2, bc2.reshape(1, -1),
        g, 2000)
    return (logits, emb)
